# baseline (speedup 1.0000x reference)
import functools
import math

import jax
import jax.numpy as jnp
import numpy as np
from jax import lax
from jax.experimental import pallas as pl
from jax.experimental.pallas import tpu as pltpu
from jax.experimental.pallas import tpu_sc as plsc

_TOPK = 64
_PRIOR_SCALE = 0.5
_CLIP = 50.0
_INT_MIN = np.int32(-2**31)
_T_PAD = 1536


def _keys_body(tg_ref, tfid_t_ref, motif_ref, keys_ref):
    d = tg_ref.shape[1]
    sim = lax.dot_general(
        tg_ref[...].astype(jnp.bfloat16), tfid_t_ref[...].astype(jnp.bfloat16),
        (((1,), (0,)), ((), ())),
        preferred_element_type=jnp.float32,
    )
    sim = sim / np.float32(math.sqrt(d)) + _PRIOR_SCALE * motif_ref[...]
    sim = jnp.clip(sim, -_CLIP, _CLIP)
    b = lax.bitcast_convert_type(sim, jnp.int32)
    key = jnp.where(b < 0, b ^ jnp.int32(0x7FFFFFFF), b)
    pad = jnp.full((key.shape[0], _T_PAD - key.shape[1]), _INT_MIN, jnp.int32)
    keys_ref[...] = jnp.concatenate([key, pad], axis=1)


def _sc_topk_body(keys_hbm, tau_hbm, kbuf, hist, taubuf):
    nc = 2
    wid = lax.axis_index("s") * nc + lax.axis_index("c")
    n_chunks = tau_hbm.shape[0]
    lanes = lax.iota(jnp.int32, 16)
    ones = jnp.ones((16,), jnp.int32)
    zeros = jnp.zeros((16,), jnp.int32)

    def chunk_body(i, _):
        c = i * 32 + wid

        @pl.when(c < n_chunks)
        def _():
            pltpu.sync_copy(keys_hbm.at[pl.ds(c * 16, 16), :], kbuf)

            def row_body(r, _):
                tgt = zeros
                k = zeros + _TOPK

                for lv in range(4):
                    shift = 24 - 8 * lv
                    for j in range(16):
                        hist[pl.ds(j * 16, 16)] = zeros

                    def pass_body(j, _, lv=lv, shift=shift, tgt=tgt):
                        v = kbuf[r, pl.ds(j * 16, 16)]
                        if lv == 0:
                            bkt = (v >> 24) + 128
                            plsc.addupdate_scatter(hist, [bkt], ones)
                        else:
                            m = (v >> (shift + 8)) == tgt
                            bkt = (v >> shift) & 255
                            plsc.addupdate_scatter(hist, [bkt], ones, mask=m)
                        return 0

                    lax.fori_loop(0, 96, pass_body, 0)

                    def scan_body(jj, carry, k=k):
                        s_run, s_fin, beta, found = carry
                        j = 15 - jj
                        h = hist[pl.ds(j * 16, 16)]
                        hr = lax.rev(h, (0,))
                        cs = plsc.cumsum(hr)
                        tot = jnp.sum(h)
                        cv = (s_run + cs) >= k
                        ncross = plsc.all_reduce_population_count(cv)
                        first = plsc.all_reduce_ffs(cv)
                        use = jnp.logical_and(jnp.logical_not(found),
                                              ncross > 0)
                        bin_idx = j * 16 + 15 - first
                        prev = jnp.sum(jnp.where(lanes == first - 1, cs, 0))
                        beta = jnp.where(use, bin_idx, beta)
                        s_fin = jnp.where(use, s_run + prev, s_fin)
                        found = jnp.logical_or(found, ncross > 0)
                        s_run = s_run + tot
                        return (s_run, s_fin, beta, found)

                    _, s_fin, beta, _ = lax.fori_loop(
                        0, 16, scan_body,
                        (zeros, zeros, zeros, zeros > 0))

                    if lv == 0:
                        tgt = beta - 128
                    else:
                        tgt = tgt * 256 + beta
                    k = k - s_fin

                plsc.store_scatter(taubuf, [zeros + r], tgt, mask=lanes == 0)
                return 0

            lax.fori_loop(0, 16, row_body, 0)
            pltpu.sync_copy(taubuf, tau_hbm.at[c])

        return 0

    lax.fori_loop(0, (n_chunks + 31) // 32, chunk_body, 0)


def _finish_body(scale_ref, keys_ref, tau_ref, tfexpr_ref, attn_ref, out_ref):
    t = attn_ref.shape[1]
    keyp = keys_ref[...]
    key = keyp[:, :t]
    b = jnp.where(key < 0, key ^ jnp.int32(0x7FFFFFFF), key)
    sim = lax.bitcast_convert_type(b, jnp.float32)

    m = jnp.max(sim, axis=1, keepdims=True)
    p = jnp.exp(sim - m)
    z = jnp.sum(p, axis=1, keepdims=True)
    mask = key >= tau_ref[...]
    s = jnp.sum(jnp.where(mask, p, 0.0), axis=1, keepdims=True)
    attn = jnp.where(mask, p / (s + 1e-8 * z), 0.0)
    attn_ref[...] = attn

    ts = lax.dot_general(
        attn.astype(jnp.bfloat16), tfexpr_ref[...].astype(jnp.bfloat16),
        (((1,), (1,)), ((), ())),
        preferred_element_type=jnp.float32,
    )
    out_ref[...] = scale_ref[0, 0] * ts


def kernel(tg_emb, tf_id_emb, tf_expr, motif_mask, scale):
    g, d = tg_emb.shape
    t = tf_id_emb.shape[0]
    b = tf_expr.shape[0]
    gb = 400
    grid = (g // gb,)
    n_chunks = g // 16

    tfid_t = tf_id_emb.T
    scale_arr = jnp.asarray(scale, jnp.float32).reshape(1, 1)

    keys = pl.pallas_call(
        _keys_body,
        grid=grid,
        in_specs=[
            pl.BlockSpec((gb, d), lambda i: (i, 0)),
            pl.BlockSpec((d, t), lambda i: (0, 0)),
            pl.BlockSpec((gb, t), lambda i: (i, 0)),
        ],
        out_specs=pl.BlockSpec((gb, _T_PAD), lambda i: (i, 0)),
        out_shape=jax.ShapeDtypeStruct((g, _T_PAD), jnp.int32),
        compiler_params=pltpu.CompilerParams(
            dimension_semantics=("arbitrary",),
        ),
    )(tg_emb, tfid_t, motif_mask)

    mesh = plsc.VectorSubcoreMesh(core_axis_name="c", subcore_axis_name="s")
    sc_topk = functools.partial(
        pl.kernel,
        mesh=mesh,
        out_type=jax.ShapeDtypeStruct((n_chunks, 16), jnp.int32),
        scratch_types=[
            pltpu.VMEM((16, _T_PAD), jnp.int32),
            pltpu.VMEM((256,), jnp.int32),
            pltpu.VMEM((16,), jnp.int32),
        ],
        compiler_params=pltpu.CompilerParams(
            use_tc_tiling_on_sc=False, needs_layout_passes=False),
    )(_sc_topk_body)
    tau = sc_topk(keys)
    tau = tau.reshape(g, 1)

    attn, out_t = pl.pallas_call(
        _finish_body,
        grid=grid,
        in_specs=[
            pl.BlockSpec(memory_space=pltpu.SMEM),
            pl.BlockSpec((gb, _T_PAD), lambda i: (i, 0)),
            pl.BlockSpec((gb, 1), lambda i: (i, 0)),
            pl.BlockSpec((b, t), lambda i: (0, 0)),
        ],
        out_specs=[
            pl.BlockSpec((gb, t), lambda i: (i, 0)),
            pl.BlockSpec((gb, b), lambda i: (i, 0)),
        ],
        out_shape=[
            jax.ShapeDtypeStruct((g, t), jnp.float32),
            jax.ShapeDtypeStruct((g, b), jnp.float32),
        ],
        compiler_params=pltpu.CompilerParams(
            dimension_semantics=("arbitrary",),
        ),
    )(scale_arr, keys, tau, tf_expr)
    return (out_t.T, attn)


# speedup vs baseline: 1.0871x; 1.0871x over previous
import functools
import math

import jax
import jax.numpy as jnp
import numpy as np
from jax import lax
from jax.experimental import pallas as pl
from jax.experimental.pallas import tpu as pltpu
from jax.experimental.pallas import tpu_sc as plsc

_TOPK = 64
_PRIOR_SCALE = 0.5
_CLIP = 50.0
_INT_MIN = np.int32(-2**31)
_T_PAD = 1536


def _keys_body(tg_ref, tfid_t_ref, motif_ref, keys_ref):
    d = tg_ref.shape[1]
    sim = lax.dot_general(
        tg_ref[...].astype(jnp.bfloat16), tfid_t_ref[...].astype(jnp.bfloat16),
        (((1,), (0,)), ((), ())),
        preferred_element_type=jnp.float32,
    )
    sim = sim / np.float32(math.sqrt(d)) + _PRIOR_SCALE * motif_ref[...]
    sim = jnp.clip(sim, -_CLIP, _CLIP)
    b = lax.bitcast_convert_type(sim, jnp.int32)
    key = jnp.where(b < 0, b ^ jnp.int32(0x7FFFFFFF), b)
    pad = jnp.full((key.shape[0], _T_PAD - key.shape[1]), _INT_MIN, jnp.int32)
    keys_ref[...] = jnp.concatenate([key, pad], axis=1)


def _sc_topk_body(keys_hbm, tau_hbm, kbuf, hist, taubuf):
    nc = 2
    wid = lax.axis_index("s") * nc + lax.axis_index("c")
    n_chunks = tau_hbm.shape[0]
    lanes = lax.iota(jnp.int32, 16)
    ones = jnp.ones((16,), jnp.int32)
    zeros = jnp.zeros((16,), jnp.int32)

    def chunk_body(i, _):
        c = i * 32 + wid

        @pl.when(c < n_chunks)
        def _():
            pltpu.sync_copy(keys_hbm.at[pl.ds(c * 16, 16), :], kbuf)

            def row_body(r, _):
                tgt = jnp.int32(0)
                k = jnp.int32(_TOPK)

                for lv in range(4):
                    shift = 24 - 8 * lv
                    for j in range(16):
                        hist[pl.ds(j * 16, 16)] = zeros

                    def pass_body(jj, _, lv=lv, shift=shift, tgt=tgt):
                        for u in range(8):
                            v = kbuf[r, pl.ds(jj * 128 + u * 16, 16)]
                            if lv == 0:
                                bkt = (v >> 24) + 128
                                plsc.addupdate_scatter(hist, [bkt], ones)
                            else:
                                m = (v >> (shift + 8)) == tgt
                                bkt = (v >> shift) & 255
                                plsc.addupdate_scatter(hist, [bkt], ones,
                                                       mask=m)
                        return 0

                    lax.fori_loop(0, 12, pass_body, 0)

                    tot = [jnp.sum(hist[pl.ds(j * 16, 16)])
                           for j in range(16)]
                    s_run = jnp.int32(0)
                    found = jnp.bool_(False)
                    j_star = jnp.int32(0)
                    s_cross = jnp.int32(0)
                    for j in range(15, -1, -1):
                        cross = jnp.logical_and(jnp.logical_not(found),
                                                s_run + tot[j] >= k)
                        j_star = jnp.where(cross, j, j_star)
                        s_cross = jnp.where(cross, s_run, s_cross)
                        found = jnp.logical_or(found, cross)
                        s_run = s_run + tot[j]
                    h = hist[pl.ds(j_star * 16, 16)]
                    hr = lax.rev(h, (0,))
                    cs = plsc.cumsum(hr)
                    cv = (s_cross + cs) >= k
                    first = plsc.all_reduce_ffs(cv)
                    prev = jnp.sum(jnp.where(lanes == first - 1, cs, 0))
                    beta = j_star * 16 + 15 - jnp.sum(
                        jnp.where(lanes == 0, first, 0))
                    s_fin = s_cross + prev

                    if lv == 0:
                        tgt = beta - 128
                    else:
                        tgt = tgt * 256 + beta
                    k = k - s_fin

                plsc.store_scatter(taubuf, [zeros + r], zeros + tgt,
                                   mask=lanes == 0)
                return 0

            lax.fori_loop(0, 16, row_body, 0)
            pltpu.sync_copy(taubuf, tau_hbm.at[c])

        return 0

    lax.fori_loop(0, (n_chunks + 31) // 32, chunk_body, 0)


def _finish_body(scale_ref, keys_ref, tau_ref, tfexpr_ref, attn_ref, out_ref):
    t = attn_ref.shape[1]
    keyp = keys_ref[...]
    key = keyp[:, :t]
    b = jnp.where(key < 0, key ^ jnp.int32(0x7FFFFFFF), key)
    sim = lax.bitcast_convert_type(b, jnp.float32)

    m = jnp.max(sim, axis=1, keepdims=True)
    p = jnp.exp(sim - m)
    z = jnp.sum(p, axis=1, keepdims=True)
    mask = key >= tau_ref[...]
    s = jnp.sum(jnp.where(mask, p, 0.0), axis=1, keepdims=True)
    attn = jnp.where(mask, p / (s + 1e-8 * z), 0.0)
    attn_ref[...] = attn

    ts = lax.dot_general(
        attn.astype(jnp.bfloat16), tfexpr_ref[...].astype(jnp.bfloat16),
        (((1,), (1,)), ((), ())),
        preferred_element_type=jnp.float32,
    )
    out_ref[...] = scale_ref[0, 0] * ts


def kernel(tg_emb, tf_id_emb, tf_expr, motif_mask, scale):
    g, d = tg_emb.shape
    t = tf_id_emb.shape[0]
    b = tf_expr.shape[0]
    gb = 400
    grid = (g // gb,)
    n_chunks = g // 16

    tfid_t = tf_id_emb.T
    scale_arr = jnp.asarray(scale, jnp.float32).reshape(1, 1)

    keys = pl.pallas_call(
        _keys_body,
        grid=grid,
        in_specs=[
            pl.BlockSpec((gb, d), lambda i: (i, 0)),
            pl.BlockSpec((d, t), lambda i: (0, 0)),
            pl.BlockSpec((gb, t), lambda i: (i, 0)),
        ],
        out_specs=pl.BlockSpec((gb, _T_PAD), lambda i: (i, 0)),
        out_shape=jax.ShapeDtypeStruct((g, _T_PAD), jnp.int32),
        compiler_params=pltpu.CompilerParams(
            dimension_semantics=("arbitrary",),
        ),
    )(tg_emb, tfid_t, motif_mask)

    mesh = plsc.VectorSubcoreMesh(core_axis_name="c", subcore_axis_name="s")
    sc_topk = functools.partial(
        pl.kernel,
        mesh=mesh,
        out_type=jax.ShapeDtypeStruct((n_chunks, 16), jnp.int32),
        scratch_types=[
            pltpu.VMEM((16, _T_PAD), jnp.int32),
            pltpu.VMEM((256,), jnp.int32),
            pltpu.VMEM((16,), jnp.int32),
        ],
        compiler_params=pltpu.CompilerParams(
            use_tc_tiling_on_sc=False, needs_layout_passes=False),
    )(_sc_topk_body)
    tau = sc_topk(keys)
    tau = tau.reshape(g, 1)

    attn, out_t = pl.pallas_call(
        _finish_body,
        grid=grid,
        in_specs=[
            pl.BlockSpec(memory_space=pltpu.SMEM),
            pl.BlockSpec((gb, _T_PAD), lambda i: (i, 0)),
            pl.BlockSpec((gb, 1), lambda i: (i, 0)),
            pl.BlockSpec((b, t), lambda i: (0, 0)),
        ],
        out_specs=[
            pl.BlockSpec((gb, t), lambda i: (i, 0)),
            pl.BlockSpec((gb, b), lambda i: (i, 0)),
        ],
        out_shape=[
            jax.ShapeDtypeStruct((g, t), jnp.float32),
            jax.ShapeDtypeStruct((g, b), jnp.float32),
        ],
        compiler_params=pltpu.CompilerParams(
            dimension_semantics=("arbitrary",),
        ),
    )(scale_arr, keys, tau, tf_expr)
    return (out_t.T, attn)


# speedup vs baseline: 2.7814x; 2.5587x over previous
import math

import jax
import jax.numpy as jnp
import numpy as np
from jax import lax
from jax.experimental import pallas as pl
from jax.experimental.pallas import tpu as pltpu

_TOPK = 64
_PRIOR_SCALE = 0.5
_CLIP = 50.0


def _sortable_key_const(x: float) -> int:
    b = np.float32(x).view(np.int32)
    if b < 0:
        b = np.int32(b ^ np.int32(0x7FFFFFFF))
    return int(b)


_LO0 = _sortable_key_const(-_CLIP)
_HI0 = _sortable_key_const(_CLIP) + 1
_BISECT_ITERS = 32


def _fused_body(scale_ref, tg_ref, tfid_t_ref, tfexpr_ref, motif_ref,
                attn_ref, out_ref):
    d = tg_ref.shape[1]
    sim = lax.dot_general(
        tg_ref[...].astype(jnp.bfloat16), tfid_t_ref[...].astype(jnp.bfloat16),
        (((1,), (0,)), ((), ())),
        preferred_element_type=jnp.float32,
    )
    sim = sim / np.float32(math.sqrt(d)) + _PRIOR_SCALE * motif_ref[...]
    sim = jnp.clip(sim, -_CLIP, _CLIP)

    m = jnp.max(sim, axis=1, keepdims=True)
    p = jnp.exp(sim - m)
    z = jnp.sum(p, axis=1, keepdims=True)

    b = lax.bitcast_convert_type(sim, jnp.int32)
    key = jnp.where(b < 0, b ^ jnp.int32(0x7FFFFFFF), b)
    lo = jnp.full((sim.shape[0], 1), _LO0, dtype=jnp.int32)
    hi = jnp.full((sim.shape[0], 1), _HI0, dtype=jnp.int32)

    cnt0 = jnp.full((sim.shape[0], 1), key.shape[1], dtype=jnp.int32)

    def cond(carry):
        i, lo, hi, cl = carry
        return jnp.logical_and(i < _BISECT_ITERS,
                               jnp.logical_not(jnp.all(cl == _TOPK)))

    def bisect(carry):
        i, lo, hi, cl = carry
        mid = (lo & hi) + ((lo ^ hi) >> 1)
        cnt = jnp.sum((key >= mid).astype(jnp.int32), axis=1, keepdims=True)
        ge = cnt >= _TOPK
        return (i + 1, jnp.where(ge, mid, lo), jnp.where(ge, hi, mid),
                jnp.where(ge, cnt, cl))

    _, lo, hi, _ = lax.while_loop(cond, bisect, (0, lo, hi, cnt0))
    mask = key >= lo

    s = jnp.sum(jnp.where(mask, p, 0.0), axis=1, keepdims=True)
    attn = jnp.where(mask, p / (s + 1e-8 * z), 0.0)
    attn_ref[...] = attn

    ts = lax.dot_general(
        attn.astype(jnp.bfloat16), tfexpr_ref[...].astype(jnp.bfloat16),
        (((1,), (1,)), ((), ())),
        preferred_element_type=jnp.float32,
    )
    out_ref[...] = scale_ref[0, 0] * ts


def kernel(tg_emb, tf_id_emb, tf_expr, motif_mask, scale):
    g, d = tg_emb.shape
    t = tf_id_emb.shape[0]
    b = tf_expr.shape[0]
    gb = 200
    grid = (g // gb,)

    tfid_t = tf_id_emb.T
    scale_arr = jnp.asarray(scale, jnp.float32).reshape(1, 1)

    attn, out_t = pl.pallas_call(
        _fused_body,
        grid=grid,
        in_specs=[
            pl.BlockSpec(memory_space=pltpu.SMEM),
            pl.BlockSpec((gb, d), lambda i: (i, 0)),
            pl.BlockSpec((d, t), lambda i: (0, 0)),
            pl.BlockSpec((b, t), lambda i: (0, 0)),
            pl.BlockSpec((gb, t), lambda i: (i, 0)),
        ],
        out_specs=[
            pl.BlockSpec((gb, t), lambda i: (i, 0)),
            pl.BlockSpec((gb, b), lambda i: (i, 0)),
        ],
        out_shape=[
            jax.ShapeDtypeStruct((g, t), jnp.float32),
            jax.ShapeDtypeStruct((g, b), jnp.float32),
        ],
        compiler_params=pltpu.CompilerParams(
            dimension_semantics=("arbitrary",),
        ),
    )(scale_arr, tg_emb, tfid_t, tf_expr, motif_mask)
    return (out_t.T, attn)


# speedup vs baseline: 3.1468x; 1.1314x over previous
import math

import jax
import jax.numpy as jnp
import numpy as np
from jax import lax
from jax.experimental import pallas as pl
from jax.experimental.pallas import tpu as pltpu

_TOPK = 64
_PRIOR_SCALE = 0.5
_CLIP = 50.0


def _sortable_key_const(x: float) -> int:
    b = np.float32(x).view(np.int32)
    if b < 0:
        b = np.int32(b ^ np.int32(0x7FFFFFFF))
    return int(b)


_LO0 = _sortable_key_const(-_CLIP)
_HI0 = _sortable_key_const(_CLIP) + 1
_BISECT_ITERS = 32


def _fused_body(scale_ref, tg_ref, tfid_t_ref, tfexpr_ref, motif_ref,
                attn_ref, out_ref):
    d = tg_ref.shape[1]
    sim = lax.dot_general(
        tg_ref[...].astype(jnp.bfloat16), tfid_t_ref[...].astype(jnp.bfloat16),
        (((1,), (0,)), ((), ())),
        preferred_element_type=jnp.float32,
    )
    sim = sim / np.float32(math.sqrt(d)) + _PRIOR_SCALE * motif_ref[...]
    sim = jnp.clip(sim, -_CLIP, _CLIP)

    m = jnp.max(sim, axis=1, keepdims=True)
    p = jnp.exp(sim - m)
    z = jnp.sum(p, axis=1, keepdims=True)

    b = lax.bitcast_convert_type(sim, jnp.int32)
    key = jnp.where(b < 0, b ^ jnp.int32(0x7FFFFFFF), b)
    lo = jnp.full((sim.shape[0], 1), _LO0, dtype=jnp.int32)
    hi = jnp.full((sim.shape[0], 1), _HI0, dtype=jnp.int32)

    cnt0 = jnp.full((sim.shape[0], 1), key.shape[1], dtype=jnp.int32)

    def cond(carry):
        i, lo, hi, cl = carry
        return jnp.logical_and(i < _BISECT_ITERS,
                               jnp.logical_not(jnp.all(cl == _TOPK)))

    def bisect(carry):
        i, lo, hi, cl = carry
        mid = (lo & hi) + ((lo ^ hi) >> 1)
        cnt = jnp.sum((key >= mid).astype(jnp.int32), axis=1, keepdims=True)
        ge = cnt >= _TOPK
        return (i + 1, jnp.where(ge, mid, lo), jnp.where(ge, hi, mid),
                jnp.where(ge, cnt, cl))

    _, lo, hi, _ = lax.while_loop(cond, bisect, (0, lo, hi, cnt0))
    mask = key >= lo

    s = jnp.sum(jnp.where(mask, p, 0.0), axis=1, keepdims=True)
    attn = jnp.where(mask, p / (s + 1e-8 * z), 0.0)
    attn_ref[...] = attn

    ts = lax.dot_general(
        attn.astype(jnp.bfloat16), tfexpr_ref[...].astype(jnp.bfloat16),
        (((1,), (1,)), ((), ())),
        preferred_element_type=jnp.float32,
    )
    out_ref[...] = scale_ref[0, 0] * ts


def kernel(tg_emb, tf_id_emb, tf_expr, motif_mask, scale):
    g, d = tg_emb.shape
    t = tf_id_emb.shape[0]
    b = tf_expr.shape[0]
    gb = 800
    grid = (g // gb,)

    tfid_t = tf_id_emb.T
    scale_arr = jnp.asarray(scale, jnp.float32).reshape(1, 1)

    attn, out_t = pl.pallas_call(
        _fused_body,
        grid=grid,
        in_specs=[
            pl.BlockSpec(memory_space=pltpu.SMEM),
            pl.BlockSpec((gb, d), lambda i: (i, 0)),
            pl.BlockSpec((d, t), lambda i: (0, 0)),
            pl.BlockSpec((b, t), lambda i: (0, 0)),
            pl.BlockSpec((gb, t), lambda i: (i, 0)),
        ],
        out_specs=[
            pl.BlockSpec((gb, t), lambda i: (i, 0)),
            pl.BlockSpec((gb, b), lambda i: (i, 0)),
        ],
        out_shape=[
            jax.ShapeDtypeStruct((g, t), jnp.float32),
            jax.ShapeDtypeStruct((g, b), jnp.float32),
        ],
        compiler_params=pltpu.CompilerParams(
            dimension_semantics=("arbitrary",),
        ),
    )(scale_arr, tg_emb, tfid_t, tf_expr, motif_mask)
    return (out_t.T, attn)
